# Initial kernel scaffold; baseline (speedup 1.0000x reference)
import jax
import jax.numpy as jnp
from jax.experimental import pallas as pl


def kernel(candidate_input, reference_ids_list):
    raise NotImplementedError("write your pallas kernel here")


# speedup vs baseline: 1.4699x; 1.4699x over previous
import functools

import jax
import jax.numpy as jnp
from jax import lax
from jax.experimental import pallas as pl
from jax.experimental.pallas import tpu as pltpu
from jax.experimental.pallas import tpu_sc as plsc

_V = 8192
_MAX_N = 4
_SEQ = 128
_SMOOTH = 1e-10
_PAIRS = tuple((n, j) for n in range(1, _MAX_N + 1) for j in range(n))
_NPAIR = len(_PAIRS)
_L = 16


def _sc_ref_rmax(ids):
  num_refs, seq = ids.shape
  ids_flat = jnp.reshape(ids, (num_refs * seq,))
  mesh = plsc.VectorSubcoreMesh(core_axis_name="c", subcore_axis_name="s")
  info = plsc.get_sparse_core_info()
  nc = info.num_cores

  @functools.partial(
      pl.kernel,
      out_type=jax.ShapeDtypeStruct((_NPAIR, _V), jnp.float32),
      mesh=mesh,
      compiler_params=pltpu.CompilerParams(needs_layout_passes=False),
      scratch_types=[
          pltpu.VMEM((num_refs * seq,), jnp.int32),
          pltpu.VMEM((num_refs * _V,), jnp.float32),
          pltpu.VMEM((_V,), jnp.float32),
      ],
  )
  def k(ids_hbm, out_hbm, ids_v, cnt_v, row_v):
    wid = lax.axis_index("s") * nc + lax.axis_index("c")

    @pl.when(wid < _NPAIR)
    def _():
      w = wid
      n = (1 + (w >= 1).astype(jnp.int32) + (w >= 3).astype(jnp.int32)
           + (w >= 6).astype(jnp.int32))
      j = w - n * (n - 1) // 2
      lo = j
      hi = j + seq - n

      pltpu.sync_copy(ids_hbm, ids_v)

      def zero_body(i, c):
        cnt_v[pl.ds(i * _L, _L)] = jnp.zeros((_L,), jnp.float32)
        return c

      lax.fori_loop(0, num_refs * _V // _L, zero_body, 0)

      lane = lax.broadcasted_iota(jnp.int32, (_L,), 0)
      ones = jnp.ones((_L,), jnp.float32)
      for r in range(num_refs):
        for g in range(seq // _L):
          idx = ids_v[pl.ds(r * seq + g * _L, _L)] + (r * _V)
          pos = lane + (g * _L)
          valid = (pos >= lo) & (pos <= hi)
          for l in range(_L):
            m = valid & (lane == l)
            plsc.addupdate_scatter(cnt_v, [idx], ones, mask=m)

      def max_body(i, c):
        o = i * _L
        v = cnt_v[pl.ds(o, _L)]
        for r in range(1, num_refs):
          v = jnp.maximum(v, cnt_v[pl.ds(r * _V + o, _L)])
        row_v[pl.ds(o, _L)] = v
        return c

      lax.fori_loop(0, _V // _L, max_body, 0)
      pltpu.sync_copy(row_v, out_hbm.at[w])

  return k(ids_flat)


def _tc_body(x_ref, rmax_ref, out_ref):
  x = x_ref[...]
  m = jnp.max(x, axis=1, keepdims=True)
  e = jnp.exp(x - m)
  s = jnp.sum(e, axis=1, keepdims=True)
  d = e / s

  t = jnp.sum(d, axis=0, keepdims=True)
  zero = jnp.zeros((1, _V), jnp.float32)
  first = [zero]
  last = [zero]
  for k in range(1, _MAX_N):
    first.append(first[-1] + d[k - 1:k, :])
    last.append(last[-1] + d[_SEQ - k:_SEQ - k + 1, :])
  s_t = jnp.sum(t)
  s_first = [jnp.sum(a) for a in first]
  s_last = [jnp.sum(b) for b in last]

  logp = jnp.float32(0.0)
  for n in range(1, _MAX_N + 1):
    tclip = jnp.float32(0.0)
    tcand = jnp.float32(0.0)
    for j in range(n):
      widx = _PAIRS.index((n, j))
      c = t - first[j] - last[n - 1 - j]
      r = rmax_ref[widx:widx + 1, :]
      tclip = tclip + jnp.sum(jnp.minimum(c, r))
      tcand = tcand + (s_t - s_first[j] - s_last[n - 1 - j])
    prec = (tclip + _SMOOTH) / (tcand + _SMOOTH)
    logp = logp + jnp.log(jnp.maximum(prec, _SMOOTH))
  out_ref[...] = jnp.exp(logp / _MAX_N)[None, None]


def _tc_bleu(x, rmax):
  return pl.pallas_call(
      _tc_body,
      out_shape=jax.ShapeDtypeStruct((1, 1), jnp.float32),
  )(x, rmax)


@jax.jit
def kernel(candidate_input, reference_ids_list):
  rmax = _sc_ref_rmax(reference_ids_list)
  return _tc_bleu(candidate_input, rmax)[0, 0]


# speedup vs baseline: 1.9099x; 1.2993x over previous
import functools

import jax
import jax.numpy as jnp
from jax import lax
from jax.experimental import pallas as pl
from jax.experimental.pallas import tpu as pltpu
from jax.experimental.pallas import tpu_sc as plsc

_V = 8192
_MAX_N = 4
_SEQ = 128
_SMOOTH = 1e-10
_L = 16


def _sc_ref_hist(ids):
  num_refs, seq = ids.shape
  ids_flat = jnp.reshape(ids, (num_refs * seq,))
  zeros = jnp.zeros((_V,), jnp.float32)
  mesh = plsc.VectorSubcoreMesh(core_axis_name="c", subcore_axis_name="s")
  info = plsc.get_sparse_core_info()
  nc = info.num_cores

  @functools.partial(
      pl.kernel,
      out_type=jax.ShapeDtypeStruct((num_refs, _V), jnp.float32),
      mesh=mesh,
      compiler_params=pltpu.CompilerParams(needs_layout_passes=False),
      scratch_types=[
          pltpu.VMEM((seq,), jnp.int32),
          pltpu.VMEM((_V,), jnp.float32),
      ],
  )
  def k(ids_hbm, zero_hbm, out_hbm, ids_v, cnt_v):
    wid = lax.axis_index("s") * nc + lax.axis_index("c")

    @pl.when(wid < num_refs)
    def _():
      r = wid
      pltpu.sync_copy(ids_hbm.at[pl.ds(r * seq, seq)], ids_v)
      pltpu.sync_copy(zero_hbm, cnt_v)

      lane = lax.broadcasted_iota(jnp.int32, (_L,), 0)
      ones = jnp.ones((_L,), jnp.float32)
      for g in range(seq // _L):
        idx = ids_v[pl.ds(g * _L, _L)]
        for l in range(_L):
          plsc.addupdate_scatter(cnt_v, [idx], ones, mask=(lane == l))

      pltpu.sync_copy(cnt_v, out_hbm.at[r])

  return k(ids_flat, zeros)


def _tc_body(x_ref, fc_ref, bnd_ref, out_ref):
  x = x_ref[...]
  m = jnp.max(x, axis=1, keepdims=True)
  e = jnp.exp(x - m)
  s = jnp.sum(e, axis=1, keepdims=True)
  d = e / s

  t = jnp.sum(d, axis=0, keepdims=True)
  zero = jnp.zeros((1, _V), jnp.float32)
  first = [zero]
  last = [zero]
  for k in range(1, _MAX_N):
    first.append(first[-1] + d[k - 1:k, :])
    last.append(last[-1] + d[_SEQ - k:_SEQ - k + 1, :])
  s_t = jnp.sum(t)
  s_first = [jnp.sum(a) for a in first]
  s_last = [jnp.sum(b) for b in last]

  num_refs = fc_ref.shape[0]
  iota = lax.broadcasted_iota(jnp.int32, (1, _V), 1)
  g = []
  h = []
  for r in range(num_refs):
    gr = [zero]
    hr = [zero]
    for k in range(1, _MAX_N):
      gr.append(gr[-1] + (iota == bnd_ref[r, k - 1]).astype(jnp.float32))
      hr.append(hr[-1] + (iota == bnd_ref[r, 2 * _MAX_N - 2 - k])
                .astype(jnp.float32))
    g.append(gr)
    h.append(hr)

  logp = jnp.float32(0.0)
  for n in range(1, _MAX_N + 1):
    tclip = jnp.float32(0.0)
    tcand = jnp.float32(0.0)
    for j in range(n):
      c = t - first[j] - last[n - 1 - j]
      rmax = None
      for r in range(num_refs):
        rc = fc_ref[r:r + 1, :] - g[r][j] - h[r][n - 1 - j]
        rmax = rc if rmax is None else jnp.maximum(rmax, rc)
      tclip = tclip + jnp.sum(jnp.minimum(c, rmax))
      tcand = tcand + (s_t - s_first[j] - s_last[n - 1 - j])
    prec = (tclip + _SMOOTH) / (tcand + _SMOOTH)
    logp = logp + jnp.log(jnp.maximum(prec, _SMOOTH))
  out_ref[...] = jnp.exp(logp / _MAX_N)[None, None]


def _tc_bleu(x, fc, bnd):
  return pl.pallas_call(
      _tc_body,
      in_specs=[
          pl.BlockSpec(memory_space=pltpu.VMEM),
          pl.BlockSpec(memory_space=pltpu.VMEM),
          pl.BlockSpec(memory_space=pltpu.SMEM),
      ],
      out_shape=jax.ShapeDtypeStruct((1, 1), jnp.float32),
  )(x, fc, bnd)


@jax.jit
def kernel(candidate_input, reference_ids_list):
  fc = _sc_ref_hist(reference_ids_list)
  bnd = jnp.concatenate(
      [reference_ids_list[:, :_MAX_N - 1],
       reference_ids_list[:, _SEQ - (_MAX_N - 1):]], axis=1)
  return _tc_bleu(candidate_input, fc, bnd)[0, 0]


# speedup vs baseline: 2.1192x; 1.1096x over previous
import functools

import jax
import jax.numpy as jnp
from jax import lax
from jax.experimental import pallas as pl
from jax.experimental.pallas import tpu as pltpu
from jax.experimental.pallas import tpu_sc as plsc

_V = 8192
_MAX_N = 4
_SEQ = 128
_SMOOTH = 1e-10
_L = 16


def _sc_ref_hist(ids):
  num_refs, seq = ids.shape
  mesh = plsc.VectorSubcoreMesh(core_axis_name="c", subcore_axis_name="s")
  info = plsc.get_sparse_core_info()
  nc = info.num_cores

  @functools.partial(
      pl.kernel,
      out_type=jax.ShapeDtypeStruct((num_refs, _V), jnp.float32),
      mesh=mesh,
      compiler_params=pltpu.CompilerParams(needs_layout_passes=False),
      scratch_types=[
          pltpu.VMEM((seq,), jnp.int32),
          pltpu.VMEM((_V,), jnp.float32),
      ],
  )
  def k(ids_hbm, out_hbm, ids_v, cnt_v):
    wid = lax.axis_index("s") * nc + lax.axis_index("c")

    @pl.when(wid < num_refs)
    def _():
      r = wid
      pltpu.sync_copy(ids_hbm.at[r], ids_v)

      def zero_body(i, c):
        z = jnp.zeros((_L,), jnp.float32)
        for u in range(4):
          cnt_v[pl.ds(i * 4 * _L + u * _L, _L)] = z
        return c

      lax.fori_loop(0, _V // (4 * _L), zero_body, 0)

      lane = lax.broadcasted_iota(jnp.int32, (_L,), 0)
      ones = jnp.ones((_L,), jnp.float32)
      for g in range(seq // _L):
        idx = ids_v[pl.ds(g * _L, _L)]
        for l in range(_L):
          plsc.addupdate_scatter(cnt_v, [idx], ones, mask=(lane == l))

      pltpu.sync_copy(cnt_v, out_hbm.at[r])

  return k(ids)


def _tc1_body(x_ref, out_ref):
  x = x_ref[...]
  m = jnp.max(x, axis=1, keepdims=True)
  e = jnp.exp(x - m)
  s = jnp.sum(e, axis=1, keepdims=True)
  d = e / s

  t = jnp.sum(d, axis=0, keepdims=True)
  rows = [t]
  acc = jnp.zeros((1, _V), jnp.float32)
  for k in range(1, _MAX_N):
    acc = acc + d[k - 1:k, :]
    rows.append(acc)
  acc = jnp.zeros((1, _V), jnp.float32)
  for k in range(1, _MAX_N):
    acc = acc + d[_SEQ - k:_SEQ - k + 1, :]
    rows.append(acc)
  rows.append(jnp.zeros((1, _V), jnp.float32))
  out_ref[...] = jnp.concatenate(rows, axis=0)


def _tc1_colsums(x):
  return pl.pallas_call(
      _tc1_body,
      out_shape=jax.ShapeDtypeStruct((2 * _MAX_N, _V), jnp.float32),
  )(x)


def _tc2_body(v_ref, fc_ref, ids_ref, out_ref):
  t = v_ref[0:1, :]
  first = [jnp.zeros((1, _V), jnp.float32)] + [
      v_ref[k:k + 1, :] for k in range(1, _MAX_N)]
  last = [jnp.zeros((1, _V), jnp.float32)] + [
      v_ref[_MAX_N - 1 + k:_MAX_N + k, :] for k in range(1, _MAX_N)]
  s_t = jnp.sum(t)
  s_first = [jnp.sum(a) for a in first]
  s_last = [jnp.sum(b) for b in last]

  num_refs = fc_ref.shape[0]
  zero = jnp.zeros((1, _V), jnp.float32)
  iota = lax.broadcasted_iota(jnp.int32, (1, _V), 1)
  g = []
  h = []
  for r in range(num_refs):
    gr = [zero]
    hr = [zero]
    for k in range(1, _MAX_N):
      gr.append(gr[-1] + (iota == ids_ref[r, k - 1]).astype(jnp.float32))
      hr.append(hr[-1] + (iota == ids_ref[r, _SEQ - k]).astype(jnp.float32))
    g.append(gr)
    h.append(hr)

  logp = jnp.float32(0.0)
  for n in range(1, _MAX_N + 1):
    tclip = jnp.float32(0.0)
    tcand = jnp.float32(0.0)
    for j in range(n):
      c = t - first[j] - last[n - 1 - j]
      rmax = None
      for r in range(num_refs):
        rc = fc_ref[r:r + 1, :] - g[r][j] - h[r][n - 1 - j]
        rmax = rc if rmax is None else jnp.maximum(rmax, rc)
      tclip = tclip + jnp.sum(jnp.minimum(c, rmax))
      tcand = tcand + (s_t - s_first[j] - s_last[n - 1 - j])
    prec = (tclip + _SMOOTH) / (tcand + _SMOOTH)
    logp = logp + jnp.log(jnp.maximum(prec, _SMOOTH))
  out_ref[...] = jnp.exp(logp / _MAX_N)[None, None]


def _tc2_combine(v, fc, ids):
  return pl.pallas_call(
      _tc2_body,
      in_specs=[
          pl.BlockSpec(memory_space=pltpu.VMEM),
          pl.BlockSpec(memory_space=pltpu.VMEM),
          pl.BlockSpec(memory_space=pltpu.SMEM),
      ],
      out_shape=jax.ShapeDtypeStruct((1, 1), jnp.float32),
  )(v, fc, ids)


@jax.jit
def kernel(candidate_input, reference_ids_list):
  fc = _sc_ref_hist(reference_ids_list)
  v = _tc1_colsums(candidate_input)
  return _tc2_combine(v, fc, reference_ids_list)[0, 0]
